# baseline (speedup 1.0000x reference)
import functools

import jax
import jax.numpy as jnp
from jax import lax
from jax.experimental import pallas as pl
from jax.experimental.pallas import tpu as pltpu
from jax.experimental.pallas import tpu_sc as plsc

V = 1000000
D = 32
B = 4096
S = 200
N = B * S

NC = 2
NS = 16
NW = NC * NS

BW = B // NW
SCH = 8
CHUNKS = S // SCH
ROWS = SCH * BW

_mesh = plsc.VectorSubcoreMesh(core_axis_name="c", subcore_axis_name="s")

BLK = 64
NBLK = V // BLK
BASE = NBLK // NW
LEFT = NBLK % NW


@functools.partial(
    pl.kernel,
    out_type=(
        jax.ShapeDtypeStruct((V * D,), jnp.float32),
        jax.ShapeDtypeStruct((V * D,), jnp.float32),
    ),
    mesh=_mesh,
    compiler_params=pltpu.CompilerParams(use_tc_tiling_on_sc=True,
                                         needs_layout_passes=False),
    scratch_types=[
        pltpu.VMEM((BLK, D), jnp.float32),
        pltpu.VMEM((BLK * D,), jnp.float32),
    ],
)
def _sc_detile(wr, wi, out_r, out_i, va, vb):
    wid = lax.axis_index("s") * NC + lax.axis_index("c")
    nblk = BASE + (wid < LEFT).astype(jnp.int32)
    start = wid * BASE + jnp.minimum(wid, LEFT)

    def make_blk(w, out):
        def blk(t, carry):
            bid = start + t
            pltpu.sync_copy(w.at[pl.ds(bid * BLK, BLK), :], va)
            for g in range(BLK * D // 16):
                vb[pl.ds(g * 16, 16)] = va[g // 2, pl.ds((g % 2) * 16, 16)]
            pltpu.sync_copy(vb, out.at[pl.ds(bid * BLK * D, BLK * D)])
            return carry
        return blk

    lax.fori_loop(0, nblk, make_blk(wr, out_r), 0)
    lax.fori_loop(0, nblk, make_blk(wi, out_i), 0)


@functools.partial(
    pl.kernel,
    out_type=(
        jax.ShapeDtypeStruct((S, D, B), jnp.float32),
        jax.ShapeDtypeStruct((S, D, B), jnp.float32),
    ),
    mesh=_mesh,
    compiler_params=pltpu.CompilerParams(use_tc_tiling_on_sc=False, needs_layout_passes=False),
    scratch_types=[
        pltpu.VMEM((SCH, BW), jnp.int32),
        pltpu.VMEM((2 * ROWS, D), jnp.float32),
        pltpu.VMEM((SCH, D, BW + 1), jnp.float32),
        pltpu.SemaphoreType.DMA,
        pltpu.SemaphoreType.DMA,
    ],
)
def _sc_gather(xt, wr, wi, out_r, out_i, idx_v, rows_v, obuf,
               rsem, isem):
    wid = lax.axis_index("s") * NC + lax.axis_index("c")
    b0 = wid * BW

    lane = lax.iota(jnp.int32, 16)

    def transpose_block(base):
        def col(bb, carry):
            bv = lane * 0 + bb
            for j in range(SCH):
                jv = jnp.full((16,), j, jnp.int32)
                for dh in range(D // 16):
                    vals = rows_v[base + j * BW + bb, pl.ds(dh * 16, 16)]
                    plsc.store_scatter(obuf, [jv, dh * 16 + lane, bv], vals)
            return carry
        lax.fori_loop(0, BW, col, 0)

    def gather_rows(w, base, sem):
        for j in range(SCH):
            pltpu.async_copy(w.at[idx_v.at[j]],
                             rows_v.at[pl.ds(base + j * BW, BW)], sem)

    def drain_rows(w, base, sem):
        for j in range(SCH):
            pltpu.make_async_copy(w.at[idx_v.at[j]],
                                  rows_v.at[pl.ds(base + j * BW, BW)],
                                  sem).wait()

    def chunk(c, carry):
        s0 = c * SCH
        pltpu.sync_copy(xt.at[pl.ds(s0, SCH), pl.ds(b0, BW)], idx_v)
        gather_rows(wr, 0, rsem)
        gather_rows(wi, ROWS, isem)
        drain_rows(wr, 0, rsem)
        transpose_block(0)
        pltpu.sync_copy(obuf.at[:, :, pl.ds(0, BW)],
                        out_r.at[pl.ds(s0, SCH), :, pl.ds(b0, BW)])
        drain_rows(wi, ROWS, isem)
        transpose_block(ROWS)
        pltpu.sync_copy(obuf.at[:, :, pl.ds(0, BW)],
                        out_i.at[pl.ds(s0, SCH), :, pl.ds(b0, BW)])
        return carry

    lax.fori_loop(0, CHUNKS, chunk, 0)


def kernel(x, W_real, W_imag):
    xt = x.T
    fr, fi = _sc_detile(W_real, W_imag)
    r, i = _sc_gather(xt, fr.reshape(V, D), fi.reshape(V, D))
    r3 = jnp.transpose(r, (2, 0, 1))
    i3 = jnp.transpose(i, (2, 0, 1))
    return lax.complex(r3, i3)


# speedup vs baseline: 1.2890x; 1.2890x over previous
import functools

import jax
import jax.numpy as jnp
from jax import lax
from jax.experimental import pallas as pl
from jax.experimental.pallas import tpu as pltpu
from jax.experimental.pallas import tpu_sc as plsc

V = 1000000
D = 32
B = 4096
S = 200
N = B * S

NC = 2
NS = 16
NW = NC * NS

BW = B // NW
SCH = 8
CHUNKS = S // SCH
ROWS = SCH * BW

_mesh = plsc.VectorSubcoreMesh(core_axis_name="c", subcore_axis_name="s")


@functools.partial(
    pl.kernel,
    out_type=(
        jax.ShapeDtypeStruct((S, D, B), jnp.float32),
        jax.ShapeDtypeStruct((S, D, B), jnp.float32),
    ),
    mesh=_mesh,
    compiler_params=pltpu.CompilerParams(use_tc_tiling_on_sc=False, needs_layout_passes=False),
    scratch_types=[
        pltpu.VMEM((SCH, BW), jnp.int32),
        pltpu.VMEM((2 * ROWS, D), jnp.float32),
        pltpu.VMEM((SCH, D, BW + 1), jnp.float32),
        pltpu.SemaphoreType.DMA,
        pltpu.SemaphoreType.DMA,
    ],
)
def _sc_gather(xt, wr, wi, out_r, out_i, idx_v, rows_v, obuf,
               rsem, isem):
    wid = lax.axis_index("s") * NC + lax.axis_index("c")
    b0 = wid * BW

    lane = lax.iota(jnp.int32, 16)

    def transpose_block(base):
        def col(bb, carry):
            bv = lane * 0 + bb
            for j in range(SCH):
                jv = jnp.full((16,), j, jnp.int32)
                for dh in range(D // 16):
                    vals = rows_v[base + j * BW + bb, pl.ds(dh * 16, 16)]
                    plsc.store_scatter(obuf, [jv, dh * 16 + lane, bv], vals)
            return carry
        lax.fori_loop(0, BW, col, 0)

    def gather_rows(w, base, sem):
        for j in range(SCH):
            pltpu.async_copy(w.at[idx_v.at[j]],
                             rows_v.at[pl.ds(base + j * BW, BW)], sem)

    def drain_rows(w, base, sem):
        for j in range(SCH):
            pltpu.make_async_copy(w.at[idx_v.at[j]],
                                  rows_v.at[pl.ds(base + j * BW, BW)],
                                  sem).wait()

    def chunk(c, carry):
        s0 = c * SCH
        pltpu.sync_copy(xt.at[pl.ds(s0, SCH), pl.ds(b0, BW)], idx_v)
        gather_rows(wr, 0, rsem)
        gather_rows(wi, ROWS, isem)
        drain_rows(wr, 0, rsem)
        transpose_block(0)
        pltpu.sync_copy(obuf.at[:, :, pl.ds(0, BW)],
                        out_r.at[pl.ds(s0, SCH), :, pl.ds(b0, BW)])
        drain_rows(wi, ROWS, isem)
        transpose_block(ROWS)
        pltpu.sync_copy(obuf.at[:, :, pl.ds(0, BW)],
                        out_i.at[pl.ds(s0, SCH), :, pl.ds(b0, BW)])
        return carry

    lax.fori_loop(0, CHUNKS, chunk, 0)


def kernel(x, W_real, W_imag):
    xt = x.T
    r, i = _sc_gather(xt, W_real, W_imag)
    r3 = jnp.transpose(r, (2, 0, 1))
    i3 = jnp.transpose(i, (2, 0, 1))
    return lax.complex(r3, i3)
